# baseline (speedup 1.0000x reference)
import functools

import jax
import jax.numpy as jnp
from jax import lax
from jax.experimental import pallas as pl
from jax.experimental.pallas import tpu as pltpu
from jax.experimental.pallas import tpu_sc as plsc

_NC = 2
_NS = 16
_NW = _NC * _NS

_KB = 8
_NBUF = 2


@functools.lru_cache(maxsize=None)
def _make_gather(batch, hist, d_model):
    assert batch % (_NW * _KB * _NBUF) == 0
    b_per_w = batch // _NW
    n_super = b_per_w // (_KB * _NBUF)

    mesh = plsc.VectorSubcoreMesh(core_axis_name="c", subcore_axis_name="s")

    @functools.partial(
        pl.kernel,
        mesh=mesh,
        out_type=jax.ShapeDtypeStruct((batch, hist, d_model), jnp.float32),
        compiler_params=pltpu.CompilerParams(use_tc_tiling_on_sc=False),
        scratch_types=[
            pltpu.VMEM((_NBUF, _KB, hist), jnp.int32),
            pltpu.VMEM((_NBUF, _KB, hist, d_model), jnp.float32),
            pltpu.SemaphoreType.DMA,
            pltpu.SemaphoreType.DMA,
            pltpu.SemaphoreType.DMA,
            pltpu.SemaphoreType.DMA,
        ],
    )
    def gather_kernel(x_hbm, table_hbm, out_hbm, idx_v, rows_v, g0, g1, o0, o1):
        gsem = [g0, g1]
        osem = [o0, o1]
        wid = lax.axis_index("s") * _NC + lax.axis_index("c")
        b_base = wid * b_per_w

        def super_iter(t, carry):
            for b in range(_NBUF):
                b0 = b_base + (t * _NBUF + b) * _KB

                @pl.when(t > 0)
                def _():
                    pltpu.make_async_copy(
                        rows_v.at[b], out_hbm.at[pl.ds(b0, _KB)], osem[b]
                    ).wait()

                pltpu.sync_copy(x_hbm.at[pl.ds(b0, _KB)], idx_v.at[b])
                for j in range(_KB):
                    pltpu.async_copy(
                        table_hbm.at[idx_v.at[b].at[j]], rows_v.at[b].at[j], gsem[b]
                    )

            for b in range(_NBUF):
                b0 = b_base + (t * _NBUF + b) * _KB
                for j in range(_KB):
                    pltpu.make_async_copy(
                        table_hbm.at[idx_v.at[b].at[j]], rows_v.at[b].at[j], gsem[b]
                    ).wait()
                pltpu.async_copy(rows_v.at[b], out_hbm.at[pl.ds(b0, _KB)], osem[b])
            return carry

        lax.fori_loop(0, n_super, super_iter, 0)

        for b in range(_NBUF):
            b0 = b_base + ((n_super - 1) * _NBUF + b) * _KB
            pltpu.make_async_copy(
                rows_v.at[b], out_hbm.at[pl.ds(b0, _KB)], osem[b]
            ).wait()

    return gather_kernel


def kernel(x, table):
    batch, hist = x.shape
    d_model = table.shape[1]
    return _make_gather(batch, hist, d_model)(x.astype(jnp.int32), table)


# speedup vs baseline: 1.3574x; 1.3574x over previous
import functools

import jax
import jax.numpy as jnp
from jax import lax
from jax.experimental import pallas as pl
from jax.experimental.pallas import tpu as pltpu
from jax.experimental.pallas import tpu_sc as plsc

_NC = 2
_NS = 16
_NW = _NC * _NS

_PADD = 128
_KB = 8
_NBUF = 2


def _pad8(n):
    return (n + 7) // 8 * 8


@functools.lru_cache(maxsize=None)
def _make_gather(batch, hist, d_model):
    assert batch % (_NW * _KB * _NBUF) == 0
    b_per_w = batch // _NW
    n_super = b_per_w // (_KB * _NBUF)
    hist_p = _pad8(hist)

    mesh = plsc.VectorSubcoreMesh(core_axis_name="c", subcore_axis_name="s")

    @functools.partial(
        pl.kernel,
        mesh=mesh,
        out_type=jax.ShapeDtypeStruct((batch * hist_p, _PADD), jnp.float32),
        compiler_params=pltpu.CompilerParams(use_tc_tiling_on_sc=False),
        scratch_types=[
            pltpu.VMEM((_NBUF, _KB, hist), jnp.int32),
            pltpu.VMEM((_NBUF, _KB * hist, d_model), jnp.float32),
            pltpu.SemaphoreType.DMA,
            pltpu.SemaphoreType.DMA,
            pltpu.SemaphoreType.DMA,
            pltpu.SemaphoreType.DMA,
        ],
    )
    def gather_kernel(x_hbm, table_hbm, out_hbm, idx_v, rows_v, g0, g1, o0, o1):
        gsem = [g0, g1]
        osem = [o0, o1]
        wid = lax.axis_index("s") * _NC + lax.axis_index("c")
        b_base = wid * b_per_w

        def out_src_dst(b, b0):
            for j in range(_KB):
                yield (
                    rows_v.at[b].at[pl.ds(j * hist, hist)],
                    out_hbm.at[pl.ds((b0 + j) * hist_p, hist), pl.ds(0, d_model)],
                )

        def super_iter(t, carry):
            for b in range(_NBUF):
                b0 = b_base + (t * _NBUF + b) * _KB

                @pl.when(t > 0)
                def _():
                    for src, dst in out_src_dst(b, b0):
                        pltpu.make_async_copy(src, dst, osem[b]).wait()

                pltpu.sync_copy(x_hbm.at[pl.ds(b0, _KB)], idx_v.at[b])
                for j in range(_KB):
                    pltpu.async_copy(
                        table_hbm.at[idx_v.at[b].at[j]],
                        rows_v.at[b].at[pl.ds(j * hist, hist)],
                        gsem[b],
                    )

            for b in range(_NBUF):
                b0 = b_base + (t * _NBUF + b) * _KB
                for j in range(_KB):
                    pltpu.make_async_copy(
                        table_hbm.at[idx_v.at[b].at[j]],
                        rows_v.at[b].at[pl.ds(j * hist, hist)],
                        gsem[b],
                    ).wait()
                for src, dst in out_src_dst(b, b0):
                    pltpu.async_copy(src, dst, osem[b])
            return carry

        lax.fori_loop(0, n_super, super_iter, 0)

        for b in range(_NBUF):
            b0 = b_base + ((n_super - 1) * _NBUF + b) * _KB
            for src, dst in out_src_dst(b, b0):
                pltpu.make_async_copy(src, dst, osem[b]).wait()

    return gather_kernel


def kernel(x, table):
    batch, hist = x.shape
    d_model = table.shape[1]
    hist_p = _pad8(hist)
    out_p = _make_gather(batch, hist, d_model)(x.astype(jnp.int32), table)
    return out_p.reshape(batch, hist_p, _PADD)[:, :hist, :d_model]
